# Initial kernel scaffold; baseline (speedup 1.0000x reference)
import jax
import jax.numpy as jnp
from jax.experimental import pallas as pl


def kernel(x, edge_index, W1, b1, W2, b2):
    raise NotImplementedError("write your pallas kernel here")


# speedup vs baseline: 38.5277x; 38.5277x over previous
import functools

import jax
import jax.numpy as jnp
from jax import lax
from jax.experimental import pallas as pl
from jax.experimental.pallas import tpu as pltpu
from jax.experimental.pallas import tpu_sc as plsc

N = 100000
E = 1600000
D_IN = 128
D_HID = 16
N_CLS = 7

NC = 2
NS = 16
NW = NC * NS

CH = 128
K = 8
M = 49
EW = K * M * CH
EP = EW * NW
NCHUNK = EP // CH
PADE = EP - E

NP = 100352
NSUB = NP // NS
NWRK = NP // NW

_MESH = plsc.VectorSubcoreMesh(core_axis_name="c", subcore_axis_name="s")


def _deg_body(dst_hbm, zer_hbm, one_hbm, dega_hbm, degb_hbm,
              deg_sh, idx_v, one_v, sem):
    c = lax.axis_index("c")
    s = lax.axis_index("s")
    off = s * NSUB
    pltpu.sync_copy(zer_hbm, deg_sh.at[pl.ds(off, NSUB)])
    pltpu.sync_copy(one_hbm, one_v)
    plsc.subcore_barrier()

    cb = (c * NS + s) * (M * K)
    pltpu.sync_copy(dst_hbm.at[pl.ds(cb, M * K)], idx_v)
    def t_body(t, carry):
        for k in range(K):
            pltpu.sync_copy(one_v, deg_sh.at[idx_v.at[t * K + k]], add=True)
        return carry
    lax.fori_loop(0, M, t_body, 0)

    plsc.subcore_barrier()

    @pl.when(c == 0)
    def _():
        pltpu.sync_copy(deg_sh.at[pl.ds(off, NSUB)], dega_hbm.at[pl.ds(off, NSUB)])

    @pl.when(c == 1)
    def _():
        pltpu.sync_copy(deg_sh.at[pl.ds(off, NSUB)], degb_hbm.at[pl.ds(off, NSUB)])


def _k_deg(dst2, z1, ones):
    f = pl.kernel(
        _deg_body,
        out_type=[jax.ShapeDtypeStruct((NP,), jnp.float32),
                  jax.ShapeDtypeStruct((NP,), jnp.float32)],
        mesh=_MESH,
        scratch_types=[
            pltpu.VMEM_SHARED((NP,), jnp.float32),
            pltpu.VMEM((M * K, CH), jnp.int32),
            pltpu.VMEM((CH,), jnp.float32),
            pltpu.SemaphoreType.DMA,
        ],
    )
    return f(dst2, z1, ones)


def _dinv_body(dega_hbm, degb_hbm, dinv_hbm, va, vb, vo):
    c = lax.axis_index("c")
    s = lax.axis_index("s")
    off = (c * NS + s) * NWRK
    pltpu.sync_copy(dega_hbm.at[pl.ds(off, NWRK)], va)
    pltpu.sync_copy(degb_hbm.at[pl.ds(off, NWRK)], vb)

    def body(i, carry):
        d = va[pl.ds(i * 16, 16)] + vb[pl.ds(i * 16, 16)] + 1.0
        bits = lax.bitcast_convert_type(d, jnp.int32)
        y = lax.bitcast_convert_type(
            jnp.int32(0x5F3759DF) - lax.shift_right_logical(bits, 1), jnp.float32)
        for _ in range(3):
            y = y * (1.5 - 0.5 * d * y * y)
        vo[pl.ds(i * 16, 16)] = y
        return carry
    lax.fori_loop(0, NWRK // 16, body, 0)
    pltpu.sync_copy(vo, dinv_hbm.at[pl.ds(off, NWRK)])


def _k_dinv(dega, degb):
    f = pl.kernel(
        _dinv_body,
        out_type=jax.ShapeDtypeStruct((NP,), jnp.float32),
        mesh=_MESH,
        scratch_types=[
            pltpu.VMEM((NWRK,), jnp.float32),
            pltpu.VMEM((NWRK,), jnp.float32),
            pltpu.VMEM((NWRK,), jnp.float32),
        ],
    )
    return f(dega, degb)


def _agg_body(src_hbm, dst_hbm, g_hbm, zer_hbm, pa_hbm, pb_hbm,
              out_sh, isv, idv, rows, sem):
    c = lax.axis_index("c")
    s = lax.axis_index("s")
    off = s * NSUB
    pltpu.sync_copy(zer_hbm, out_sh.at[pl.ds(off, NSUB)])
    plsc.subcore_barrier()

    cb = (c * NS + s) * (M * K)

    def t_body(t, carry):
        pltpu.sync_copy(src_hbm.at[pl.ds(cb + t * K, K)], isv)
        pltpu.sync_copy(dst_hbm.at[pl.ds(cb + t * K, K)], idv)
        descs = [
            pltpu.async_copy(g_hbm.at[isv.at[k]], rows.at[k], sem)
            for k in range(K)
        ]
        for d in descs:
            d.wait()
        for k in range(K):
            pltpu.sync_copy(rows.at[k], out_sh.at[idv.at[k]], add=True)
        return carry
    lax.fori_loop(0, M, t_body, 0)

    plsc.subcore_barrier()

    @pl.when(c == 0)
    def _():
        pltpu.sync_copy(out_sh.at[pl.ds(off, NSUB)], pa_hbm.at[pl.ds(off, NSUB)])

    @pl.when(c == 1)
    def _():
        pltpu.sync_copy(out_sh.at[pl.ds(off, NSUB)], pb_hbm.at[pl.ds(off, NSUB)])


def _k_agg(src2, dst2, g, z16):
    f = pl.kernel(
        _agg_body,
        out_type=[jax.ShapeDtypeStruct((NP, D_HID), jnp.float32),
                  jax.ShapeDtypeStruct((NP, D_HID), jnp.float32)],
        mesh=_MESH,
        scratch_types=[
            pltpu.VMEM_SHARED((NP, D_HID), jnp.float32),
            pltpu.VMEM((K, CH), jnp.int32),
            pltpu.VMEM((K, CH), jnp.int32),
            pltpu.VMEM((K, CH, D_HID), jnp.float32),
            pltpu.SemaphoreType.DMA,
        ],
        compiler_params=pltpu.CompilerParams(use_tc_tiling_on_sc=False),
    )
    return f(src2, dst2, g, z16)


_R = 1000


def _mm1_body(x_ref, w_ref, dv_ref, o_ref):
    h = jnp.dot(x_ref[...], w_ref[...], preferred_element_type=jnp.float32)
    o_ref[...] = h * dv_ref[...]


def _k_mm1(x, W1, dv):
    return pl.pallas_call(
        _mm1_body,
        grid=(N // _R,),
        in_specs=[
            pl.BlockSpec((_R, D_IN), lambda i: (i, 0)),
            pl.BlockSpec((D_IN, D_HID), lambda i: (0, 0)),
            pl.BlockSpec((_R, 1), lambda i: (i, 0)),
        ],
        out_specs=pl.BlockSpec((_R, D_HID), lambda i: (i, 0)),
        out_shape=jax.ShapeDtypeStruct((N, D_HID), jnp.float32),
    )(x, W1, dv)


def _mm2_body(pa_ref, pb_ref, g1_ref, dv_ref, b1_ref, w2_ref, o_ref):
    agg = pa_ref[...] + pb_ref[...] + g1_ref[...]
    h = jnp.maximum(agg * dv_ref[...] + b1_ref[...], 0.0)
    o_ref[...] = jnp.dot(h, w2_ref[...],
                         preferred_element_type=jnp.float32) * dv_ref[...]


def _k_mm2(pa, pb, g1, dv, b1r, W2p):
    return pl.pallas_call(
        _mm2_body,
        grid=(N // _R,),
        in_specs=[
            pl.BlockSpec((_R, D_HID), lambda i: (i, 0)),
            pl.BlockSpec((_R, D_HID), lambda i: (i, 0)),
            pl.BlockSpec((_R, D_HID), lambda i: (i, 0)),
            pl.BlockSpec((_R, 1), lambda i: (i, 0)),
            pl.BlockSpec((1, D_HID), lambda i: (0, 0)),
            pl.BlockSpec((D_HID, D_HID), lambda i: (0, 0)),
        ],
        out_specs=pl.BlockSpec((_R, D_HID), lambda i: (i, 0)),
        out_shape=jax.ShapeDtypeStruct((N, D_HID), jnp.float32),
    )(pa, pb, g1, dv, b1r, W2p)


def _fin_body(qa_ref, qb_ref, g2_ref, dv_ref, b2_ref, o_ref):
    z = (qa_ref[...] + qb_ref[...] + g2_ref[...]) * dv_ref[...] + b2_ref[...]
    col = lax.broadcasted_iota(jnp.int32, (_R, D_HID), 1)
    valid = col < N_CLS
    m = jnp.max(jnp.where(valid, z, -1e30), axis=1, keepdims=True)
    ex = jnp.where(valid, jnp.exp(z - m), 0.0)
    lse = jnp.log(jnp.sum(ex, axis=1, keepdims=True)) + m
    o_ref[...] = z - lse


def _k_final(qa, qb, g2, dv, b2p):
    return pl.pallas_call(
        _fin_body,
        grid=(N // _R,),
        in_specs=[
            pl.BlockSpec((_R, D_HID), lambda i: (i, 0)),
            pl.BlockSpec((_R, D_HID), lambda i: (i, 0)),
            pl.BlockSpec((_R, D_HID), lambda i: (i, 0)),
            pl.BlockSpec((_R, 1), lambda i: (i, 0)),
            pl.BlockSpec((1, D_HID), lambda i: (0, 0)),
        ],
        out_specs=pl.BlockSpec((_R, D_HID), lambda i: (i, 0)),
        out_shape=jax.ShapeDtypeStruct((N, D_HID), jnp.float32),
    )(qa, qb, g2, dv, b2p)


def kernel(x, edge_index, W1, b1, W2, b2):
    src = edge_index[0].astype(jnp.int32)
    dst = edge_index[1].astype(jnp.int32)
    pad_src = (jnp.arange(PADE, dtype=jnp.int32) * 7919) % N
    pad_dst = jnp.full((PADE,), N, jnp.int32)
    src2 = jnp.concatenate([src, pad_src]).reshape(NCHUNK, CH)
    dst2 = jnp.concatenate([dst, pad_dst]).reshape(NCHUNK, CH)

    z1 = jnp.zeros((NSUB,), jnp.float32)
    z16 = jnp.zeros((NSUB, D_HID), jnp.float32)
    ones = jnp.ones((CH,), jnp.float32)

    dega, degb = _k_deg(dst2, z1, ones)
    dinv = _k_dinv(dega, degb)
    dv = dinv[:N].reshape(N, 1)

    g1 = _k_mm1(x, W1, dv)
    p1a, p1b = _k_agg(src2, dst2, g1, z16)

    W2p = jnp.pad(W2, ((0, 0), (0, D_HID - N_CLS)))
    b1r = b1.reshape(1, D_HID)
    b2p = jnp.pad(b2, (0, D_HID - N_CLS)).reshape(1, D_HID)

    g2 = _k_mm2(p1a, p1b, g1, dv, b1r, W2p)
    p2a, p2b = _k_agg(src2, dst2, g2, z16)
    out = _k_final(p2a, p2b, g2, dv, b2p)
    return out[:, :N_CLS]


# speedup vs baseline: 49.9989x; 1.2977x over previous
import jax
import jax.numpy as jnp
from jax import lax
from jax.experimental import pallas as pl
from jax.experimental.pallas import tpu as pltpu
from jax.experimental.pallas import tpu_sc as plsc

N = 100000
E = 1600000
D_IN = 128
D_HID = 16
N_CLS = 7

NC = 2
NS = 16
NW = NC * NS

CH = 128
K = 8
M = 49
EW = K * M * CH
EP = EW * NW
NCHUNK = EP // CH
PADE = EP - E

NP = 100352
PH = NP // 8
NSUB = NP // NS
NWRK = NP // NW

_MESH = plsc.VectorSubcoreMesh(core_axis_name="c", subcore_axis_name="s")
_SC_PARAMS = pltpu.CompilerParams(use_tc_tiling_on_sc=False)


def _deg_body(dst_hbm, zer_hbm, one_hbm, dega_hbm, degb_hbm,
              deg_sh, idx_v, one_v, sem):
    c = lax.axis_index("c")
    s = lax.axis_index("s")
    off = s * NSUB
    pltpu.sync_copy(zer_hbm, deg_sh.at[pl.ds(off, NSUB)])
    pltpu.sync_copy(one_hbm, one_v)
    plsc.subcore_barrier()

    cb = (c * NS + s) * (M * K)
    pltpu.sync_copy(dst_hbm.at[pl.ds(cb, M * K)], idx_v)
    def t_body(t, carry):
        for k in range(K):
            pltpu.sync_copy(one_v, deg_sh.at[idx_v.at[t * K + k]], add=True)
        return carry
    lax.fori_loop(0, M, t_body, 0)

    plsc.subcore_barrier()

    @pl.when(c == 0)
    def _():
        pltpu.sync_copy(deg_sh.at[pl.ds(off, NSUB)], dega_hbm.at[pl.ds(off, NSUB)])

    @pl.when(c == 1)
    def _():
        pltpu.sync_copy(deg_sh.at[pl.ds(off, NSUB)], degb_hbm.at[pl.ds(off, NSUB)])


def _k_deg(dst2, z1, ones):
    f = pl.kernel(
        _deg_body,
        out_type=[jax.ShapeDtypeStruct((NP,), jnp.float32),
                  jax.ShapeDtypeStruct((NP,), jnp.float32)],
        mesh=_MESH,
        scratch_types=[
            pltpu.VMEM_SHARED((NP,), jnp.float32),
            pltpu.VMEM((M * K, CH), jnp.int32),
            pltpu.VMEM((CH,), jnp.float32),
            pltpu.SemaphoreType.DMA,
        ],
        compiler_params=_SC_PARAMS,
    )
    return f(dst2, z1, ones)


def _dinv_body(dega_hbm, degb_hbm, dv_hbm, va, vb, vo, stage):
    c = lax.axis_index("c")
    s = lax.axis_index("s")
    w = c * NS + s
    off = w * NWRK
    pltpu.sync_copy(dega_hbm.at[pl.ds(off, NWRK)], va)
    pltpu.sync_copy(degb_hbm.at[pl.ds(off, NWRK)], vb)

    def body(i, carry):
        d = va[pl.ds(i * 16, 16)] + vb[pl.ds(i * 16, 16)] + 1.0
        bits = lax.bitcast_convert_type(d, jnp.int32)
        y = lax.bitcast_convert_type(
            jnp.int32(0x5F3759DF) - lax.shift_right_logical(bits, 1), jnp.float32)
        for _ in range(3):
            y = y * (1.5 - 0.5 * d * y * y)
        vo[pl.ds(i * 16, 16)] = y
        return carry
    lax.fori_loop(0, NWRK // 16, body, 0)

    def row_body(i, carry):
        v = vo[pl.ds(i * 16, 16)]
        for u in range(16):
            stage[2 * i + u // 8, pl.ds((u % 8) * 16, 16)] = jnp.full(
                (16,), v[u], jnp.float32)
        return carry
    lax.fori_loop(0, NWRK // 16, row_body, 0)
    pltpu.sync_copy(stage, dv_hbm.at[pl.ds(w * (NWRK // 8), NWRK // 8)])


def _k_dinv(dega, degb):
    f = pl.kernel(
        _dinv_body,
        out_type=jax.ShapeDtypeStruct((PH, CH), jnp.float32),
        mesh=_MESH,
        scratch_types=[
            pltpu.VMEM((NWRK,), jnp.float32),
            pltpu.VMEM((NWRK,), jnp.float32),
            pltpu.VMEM((NWRK,), jnp.float32),
            pltpu.VMEM((NWRK // 8, CH), jnp.float32),
        ],
        compiler_params=_SC_PARAMS,
    )
    return f(dega, degb)


def _agg_body(src_hbm, dst_hbm, g_hbm, zer_hbm, pa_hbm, pb_hbm,
              out_sh, isv, idv, rows, sem):
    c = lax.axis_index("c")
    s = lax.axis_index("s")
    off = s * NSUB
    pltpu.sync_copy(zer_hbm, out_sh.at[pl.ds(off, NSUB)])
    plsc.subcore_barrier()

    cb = (c * NS + s) * (M * K)

    def t_body(t, carry):
        pltpu.sync_copy(src_hbm.at[pl.ds(cb + t * K, K)], isv)
        pltpu.sync_copy(dst_hbm.at[pl.ds(cb + t * K, K)], idv)
        descs = [
            pltpu.async_copy(g_hbm.at[isv.at[k]], rows.at[k], sem)
            for k in range(K)
        ]
        for d in descs:
            d.wait()
        for k in range(K):
            pltpu.sync_copy(rows.at[k], out_sh.at[idv.at[k]], add=True)
        return carry
    lax.fori_loop(0, M, t_body, 0)

    plsc.subcore_barrier()

    @pl.when(c == 0)
    def _():
        pltpu.sync_copy(out_sh.at[pl.ds(off, NSUB)], pa_hbm.at[pl.ds(off, NSUB)])

    @pl.when(c == 1)
    def _():
        pltpu.sync_copy(out_sh.at[pl.ds(off, NSUB)], pb_hbm.at[pl.ds(off, NSUB)])


def _k_agg(src2, dst2, g, z16):
    f = pl.kernel(
        _agg_body,
        out_type=[jax.ShapeDtypeStruct((NP, D_HID), jnp.float32),
                  jax.ShapeDtypeStruct((NP, D_HID), jnp.float32)],
        mesh=_MESH,
        scratch_types=[
            pltpu.VMEM_SHARED((NP, D_HID), jnp.float32),
            pltpu.VMEM((K, CH), jnp.int32),
            pltpu.VMEM((K, CH), jnp.int32),
            pltpu.VMEM((K, CH, D_HID), jnp.float32),
            pltpu.SemaphoreType.DMA,
        ],
        compiler_params=_SC_PARAMS,
    )
    return f(src2, dst2, g, z16)


_R = 1024
_RP = _R // 8
_G = NP // _R


def _mm1_body(x_ref, w_ref, dvp_ref, o_ref):
    for u in range(8):
        h = jnp.dot(x_ref[:, u, :], w_ref[...],
                    preferred_element_type=jnp.float32)
        o_ref[:, u * D_HID:(u + 1) * D_HID] = (
            h * dvp_ref[:, u * D_HID:(u + 1) * D_HID])


def _k_mm1(x3, W1, dvp):
    return pl.pallas_call(
        _mm1_body,
        grid=(_G,),
        in_specs=[
            pl.BlockSpec((_RP, 8, D_IN), lambda i: (i, 0, 0)),
            pl.BlockSpec((D_IN, D_HID), lambda i: (0, 0)),
            pl.BlockSpec((_RP, CH), lambda i: (i, 0)),
        ],
        out_specs=pl.BlockSpec((_RP, CH), lambda i: (i, 0)),
        out_shape=jax.ShapeDtypeStruct((PH, CH), jnp.float32),
    )(x3, W1, dvp)


def _mm2_body(pa_ref, pb_ref, g1_ref, dvp_ref, b1_ref, w2_ref, o_ref):
    aggp = pa_ref[...] + pb_ref[...] + g1_ref[...]
    dvp = dvp_ref[...]
    hp = jnp.maximum(aggp * dvp + b1_ref[...], 0.0)
    for u in range(8):
        g2 = jnp.dot(hp[:, u * D_HID:(u + 1) * D_HID], w2_ref[...],
                     preferred_element_type=jnp.float32)
        o_ref[:, u * D_HID:(u + 1) * D_HID] = (
            g2 * dvp[:, u * D_HID:(u + 1) * D_HID])


def _k_mm2(pa, pb, g1p, dvp, b1p, W2p):
    return pl.pallas_call(
        _mm2_body,
        grid=(_G,),
        in_specs=[
            pl.BlockSpec((_RP, CH), lambda i: (i, 0)),
            pl.BlockSpec((_RP, CH), lambda i: (i, 0)),
            pl.BlockSpec((_RP, CH), lambda i: (i, 0)),
            pl.BlockSpec((_RP, CH), lambda i: (i, 0)),
            pl.BlockSpec((1, CH), lambda i: (0, 0)),
            pl.BlockSpec((D_HID, D_HID), lambda i: (0, 0)),
        ],
        out_specs=pl.BlockSpec((_RP, CH), lambda i: (i, 0)),
        out_shape=jax.ShapeDtypeStruct((PH, CH), jnp.float32),
    )(pa, pb, g1p, dvp, b1p, W2p)


def _fin_body(qa_ref, qb_ref, g2_ref, dvp_ref, b2_ref, o_ref):
    z = (qa_ref[...] + qb_ref[...] + g2_ref[...]) * dvp_ref[...] + b2_ref[...]
    col = lax.broadcasted_iota(jnp.int32, (_RP, CH), 1)
    valid = (col % D_HID) < N_CLS
    ex = jnp.where(valid, jnp.exp(z), 0.0)
    gi = lax.broadcasted_iota(jnp.int32, (CH, CH), 0) // D_HID
    gj = lax.broadcasted_iota(jnp.int32, (CH, CH), 1) // D_HID
    gmat = (gi == gj).astype(jnp.float32)
    ssum = jnp.dot(ex, gmat, preferred_element_type=jnp.float32)
    o_ref[...] = z - jnp.log(ssum)


def _k_final(qa, qb, g2p, dvp, b2p):
    return pl.pallas_call(
        _fin_body,
        grid=(_G,),
        in_specs=[
            pl.BlockSpec((_RP, CH), lambda i: (i, 0)),
            pl.BlockSpec((_RP, CH), lambda i: (i, 0)),
            pl.BlockSpec((_RP, CH), lambda i: (i, 0)),
            pl.BlockSpec((_RP, CH), lambda i: (i, 0)),
            pl.BlockSpec((1, CH), lambda i: (0, 0)),
        ],
        out_specs=pl.BlockSpec((_RP, CH), lambda i: (i, 0)),
        out_shape=jax.ShapeDtypeStruct((PH, CH), jnp.float32),
    )(qa, qb, g2p, dvp, b2p)


def kernel(x, edge_index, W1, b1, W2, b2):
    src = edge_index[0].astype(jnp.int32)
    dst = edge_index[1].astype(jnp.int32)
    pad_src = (jnp.arange(PADE, dtype=jnp.int32) * 7919) % N
    pad_dst = jnp.full((PADE,), N, jnp.int32)
    src2 = jnp.concatenate([src, pad_src]).reshape(NCHUNK, CH)
    dst2 = jnp.concatenate([dst, pad_dst]).reshape(NCHUNK, CH)

    z1 = jnp.zeros((NSUB,), jnp.float32)
    z16 = jnp.zeros((NSUB, D_HID), jnp.float32)
    ones = jnp.ones((CH,), jnp.float32)

    dega, degb = _k_deg(dst2, z1, ones)
    dvp = _k_dinv(dega, degb)

    x3 = x.reshape(N // 8, 8, D_IN)
    g1p = _k_mm1(x3, W1, dvp)
    p1a, p1b = _k_agg(src2, dst2, g1p.reshape(NP, D_HID), z16)

    W2p = jnp.pad(W2, ((0, 0), (0, D_HID - N_CLS)))
    b1p = jnp.tile(b1, 8).reshape(1, CH)
    b2p = jnp.tile(jnp.pad(b2, (0, D_HID - N_CLS)), 8).reshape(1, CH)

    g2p = _k_mm2(p1a.reshape(PH, CH), p1b.reshape(PH, CH), g1p, dvp, b1p, W2p)
    p2a, p2b = _k_agg(src2, dst2, g2p.reshape(NP, D_HID), z16)
    outp = _k_final(p2a.reshape(PH, CH), p2b.reshape(PH, CH), g2p, dvp, b2p)
    return outp.reshape(NP, D_HID)[:N, :N_CLS]


# speedup vs baseline: 74.2736x; 1.4855x over previous
import jax
import jax.numpy as jnp
from jax import lax
from jax.experimental import pallas as pl
from jax.experimental.pallas import tpu as pltpu
from jax.experimental.pallas import tpu_sc as plsc

N = 100000
E = 1600000
D_IN = 128
D_HID = 16
N_CLS = 7

NC = 2
NS = 16
NW = NC * NS

CH = 128
K = 4
M = 98
EW = K * M * CH
EP = EW * NW
NCHUNK = EP // CH
PADE = EP - E

NP = 100352
PH = NP // 8
NSUB = NP // NS
NWRK = NP // NW

_MESH = plsc.VectorSubcoreMesh(core_axis_name="c", subcore_axis_name="s")
_SC_PARAMS = pltpu.CompilerParams(use_tc_tiling_on_sc=False)


def _deg_body(dst_hbm, zer_hbm, one_hbm, dega_hbm, degb_hbm,
              deg_sh, idx_v, one_v, sem):
    c = lax.axis_index("c")
    s = lax.axis_index("s")
    off = s * NSUB
    pltpu.sync_copy(zer_hbm, deg_sh.at[pl.ds(off, NSUB)])
    pltpu.sync_copy(one_hbm, one_v)
    plsc.subcore_barrier()

    cb = (c * NS + s) * (M * K)
    pltpu.sync_copy(dst_hbm.at[pl.ds(cb, M * K)], idx_v)

    def t_body(t, carry):
        for k in range(K):
            pltpu.async_copy(one_v, deg_sh.at[idx_v.at[t * K + k]], sem,
                             add=True)

        @pl.when(t > 0)
        def _():
            for k in range(K):
                pltpu.make_async_copy(
                    one_v, deg_sh.at[idx_v.at[(t - 1) * K + k]], sem).wait()
        return carry
    lax.fori_loop(0, M, t_body, 0)
    for k in range(K):
        pltpu.make_async_copy(
            one_v, deg_sh.at[idx_v.at[(M - 1) * K + k]], sem).wait()

    plsc.subcore_barrier()

    @pl.when(c == 0)
    def _():
        pltpu.sync_copy(deg_sh.at[pl.ds(off, NSUB)], dega_hbm.at[pl.ds(off, NSUB)])

    @pl.when(c == 1)
    def _():
        pltpu.sync_copy(deg_sh.at[pl.ds(off, NSUB)], degb_hbm.at[pl.ds(off, NSUB)])


def _k_deg(dst2, z1, ones):
    f = pl.kernel(
        _deg_body,
        out_type=[jax.ShapeDtypeStruct((NP,), jnp.float32),
                  jax.ShapeDtypeStruct((NP,), jnp.float32)],
        mesh=_MESH,
        scratch_types=[
            pltpu.VMEM_SHARED((NP,), jnp.float32),
            pltpu.VMEM((M * K, CH), jnp.int32),
            pltpu.VMEM((CH,), jnp.float32),
            pltpu.SemaphoreType.DMA,
        ],
        compiler_params=_SC_PARAMS,
    )
    return f(dst2, z1, ones)


def _dinv_body(dega_hbm, degb_hbm, dv_hbm, va, vb, vo, stage):
    c = lax.axis_index("c")
    s = lax.axis_index("s")
    w = c * NS + s
    off = w * NWRK
    pltpu.sync_copy(dega_hbm.at[pl.ds(off, NWRK)], va)
    pltpu.sync_copy(degb_hbm.at[pl.ds(off, NWRK)], vb)

    def body(i, carry):
        d = va[pl.ds(i * 16, 16)] + vb[pl.ds(i * 16, 16)] + 1.0
        bits = lax.bitcast_convert_type(d, jnp.int32)
        y = lax.bitcast_convert_type(
            jnp.int32(0x5F3759DF) - lax.shift_right_logical(bits, 1), jnp.float32)
        for _ in range(3):
            y = y * (1.5 - 0.5 * d * y * y)
        vo[pl.ds(i * 16, 16)] = y
        return carry
    lax.fori_loop(0, NWRK // 16, body, 0)

    def row_body(i, carry):
        v = vo[pl.ds(i * 16, 16)]
        for u in range(16):
            stage[2 * i + u // 8, pl.ds((u % 8) * 16, 16)] = jnp.full(
                (16,), v[u], jnp.float32)
        return carry
    lax.fori_loop(0, NWRK // 16, row_body, 0)
    pltpu.sync_copy(stage, dv_hbm.at[pl.ds(w * (NWRK // 8), NWRK // 8)])


def _k_dinv(dega, degb):
    f = pl.kernel(
        _dinv_body,
        out_type=jax.ShapeDtypeStruct((PH, CH), jnp.float32),
        mesh=_MESH,
        scratch_types=[
            pltpu.VMEM((NWRK,), jnp.float32),
            pltpu.VMEM((NWRK,), jnp.float32),
            pltpu.VMEM((NWRK,), jnp.float32),
            pltpu.VMEM((NWRK // 8, CH), jnp.float32),
        ],
        compiler_params=_SC_PARAMS,
    )
    return f(dega, degb)


def _agg_body(src_hbm, dst_hbm, g_hbm, zer_hbm, pa_hbm, pb_hbm,
              out_sh, isv, idv, rows, isem, gs0, gs1, ss0, ss1):
    c = lax.axis_index("c")
    s = lax.axis_index("s")
    off = s * NSUB
    pltpu.sync_copy(zer_hbm, out_sh.at[pl.ds(off, NSUB)])
    plsc.subcore_barrier()

    cb = (c * NS + s) * (M * K)
    gsem = (gs0, gs1)
    ssem = (ss0, ss1)

    pltpu.sync_copy(src_hbm.at[pl.ds(cb, K)], isv.at[pl.ds(0, K)])
    pltpu.sync_copy(dst_hbm.at[pl.ds(cb, K)], idv.at[pl.ds(0, K)])
    for k in range(K):
        pltpu.async_copy(g_hbm.at[isv.at[k]], rows.at[k], gsem[0])

    def step(t, buf):
        nxt = 1 - buf
        bb = buf * K
        nb = nxt * K

        @pl.when(t >= 1)
        def _():
            for k in range(K):
                pltpu.make_async_copy(
                    rows.at[nb + k], out_sh.at[idv.at[nb + k]],
                    ssem[nxt]).wait()

        @pl.when(t + 1 < M)
        def _():
            pltpu.async_copy(src_hbm.at[pl.ds(cb + (t + 1) * K, K)],
                             isv.at[pl.ds(nb, K)], isem)
            pltpu.async_copy(dst_hbm.at[pl.ds(cb + (t + 1) * K, K)],
                             idv.at[pl.ds(nb, K)], isem)
            pltpu.make_async_copy(src_hbm.at[pl.ds(cb, K)],
                                  isv.at[pl.ds(nb, K)], isem).wait()
            pltpu.make_async_copy(dst_hbm.at[pl.ds(cb, K)],
                                  idv.at[pl.ds(nb, K)], isem).wait()
            for k in range(K):
                pltpu.async_copy(g_hbm.at[isv.at[nb + k]], rows.at[nb + k],
                                 gsem[nxt])

        for k in range(K):
            pltpu.make_async_copy(g_hbm.at[isv.at[bb + k]], rows.at[bb + k],
                                  gsem[buf]).wait()
        for k in range(K):
            pltpu.async_copy(rows.at[bb + k], out_sh.at[idv.at[bb + k]],
                             ssem[buf], add=True)

    def pair_body(i, carry):
        step(2 * i, 0)
        step(2 * i + 1, 1)
        return carry
    lax.fori_loop(0, M // 2, pair_body, 0)
    for k in range(K):
        pltpu.make_async_copy(rows.at[K + k], out_sh.at[idv.at[K + k]],
                              ssem[1]).wait()

    plsc.subcore_barrier()

    @pl.when(c == 0)
    def _():
        pltpu.sync_copy(out_sh.at[pl.ds(off, NSUB)], pa_hbm.at[pl.ds(off, NSUB)])

    @pl.when(c == 1)
    def _():
        pltpu.sync_copy(out_sh.at[pl.ds(off, NSUB)], pb_hbm.at[pl.ds(off, NSUB)])


def _k_agg(src2, dst2, g, z16):
    f = pl.kernel(
        _agg_body,
        out_type=[jax.ShapeDtypeStruct((NP, D_HID), jnp.float32),
                  jax.ShapeDtypeStruct((NP, D_HID), jnp.float32)],
        mesh=_MESH,
        scratch_types=[
            pltpu.VMEM_SHARED((NP, D_HID), jnp.float32),
            pltpu.VMEM((2 * K, CH), jnp.int32),
            pltpu.VMEM((2 * K, CH), jnp.int32),
            pltpu.VMEM((2 * K, CH, D_HID), jnp.float32),
            pltpu.SemaphoreType.DMA,
            pltpu.SemaphoreType.DMA,
            pltpu.SemaphoreType.DMA,
            pltpu.SemaphoreType.DMA,
            pltpu.SemaphoreType.DMA,
        ],
        compiler_params=_SC_PARAMS,
    )
    return f(src2, dst2, g, z16)


_R = 2048
_RP = _R // 8
_G = NP // _R


def _mm1_body(x_ref, w_ref, dvp_ref, o_ref):
    for u in range(8):
        h = jnp.dot(x_ref[:, u, :], w_ref[...],
                    preferred_element_type=jnp.float32)
        o_ref[:, u * D_HID:(u + 1) * D_HID] = (
            h * dvp_ref[:, u * D_HID:(u + 1) * D_HID])


def _k_mm1(x3, W1, dvp):
    return pl.pallas_call(
        _mm1_body,
        grid=(_G,),
        in_specs=[
            pl.BlockSpec((_RP, 8, D_IN), lambda i: (i, 0, 0)),
            pl.BlockSpec((D_IN, D_HID), lambda i: (0, 0)),
            pl.BlockSpec((_RP, CH), lambda i: (i, 0)),
        ],
        out_specs=pl.BlockSpec((_RP, CH), lambda i: (i, 0)),
        out_shape=jax.ShapeDtypeStruct((PH, CH), jnp.float32),
    )(x3, W1, dvp)


def _mm2_body(pa_ref, pb_ref, g1_ref, dvp_ref, b1_ref, w2t_ref, o_ref):
    aggp = pa_ref[...] + pb_ref[...] + g1_ref[...]
    dvp = dvp_ref[...]
    hp = jnp.maximum(aggp * dvp + b1_ref[...], 0.0)
    g2 = jnp.dot(hp, w2t_ref[...], preferred_element_type=jnp.float32)
    o_ref[...] = g2 * dvp


def _k_mm2(pa, pb, g1p, dvp, b1p, W2t):
    return pl.pallas_call(
        _mm2_body,
        grid=(_G,),
        in_specs=[
            pl.BlockSpec((_RP, CH), lambda i: (i, 0)),
            pl.BlockSpec((_RP, CH), lambda i: (i, 0)),
            pl.BlockSpec((_RP, CH), lambda i: (i, 0)),
            pl.BlockSpec((_RP, CH), lambda i: (i, 0)),
            pl.BlockSpec((1, CH), lambda i: (0, 0)),
            pl.BlockSpec((CH, CH), lambda i: (0, 0)),
        ],
        out_specs=pl.BlockSpec((_RP, CH), lambda i: (i, 0)),
        out_shape=jax.ShapeDtypeStruct((PH, CH), jnp.float32),
    )(pa, pb, g1p, dvp, b1p, W2t)


def _fin_body(qa_ref, qb_ref, g2_ref, dvp_ref, b2_ref, o_ref):
    z = (qa_ref[...] + qb_ref[...] + g2_ref[...]) * dvp_ref[...] + b2_ref[...]
    col = lax.broadcasted_iota(jnp.int32, (_RP, CH), 1)
    valid = (col % D_HID) < N_CLS
    ex = jnp.where(valid, jnp.exp(z), 0.0)
    gi = lax.broadcasted_iota(jnp.int32, (CH, CH), 0) // D_HID
    gj = lax.broadcasted_iota(jnp.int32, (CH, CH), 1) // D_HID
    gmat = (gi == gj).astype(jnp.float32)
    ssum = jnp.dot(ex, gmat, preferred_element_type=jnp.float32)
    o_ref[...] = z - jnp.log(ssum)


def _k_final(qa, qb, g2p, dvp, b2p):
    return pl.pallas_call(
        _fin_body,
        grid=(_G,),
        in_specs=[
            pl.BlockSpec((_RP, CH), lambda i: (i, 0)),
            pl.BlockSpec((_RP, CH), lambda i: (i, 0)),
            pl.BlockSpec((_RP, CH), lambda i: (i, 0)),
            pl.BlockSpec((_RP, CH), lambda i: (i, 0)),
            pl.BlockSpec((1, CH), lambda i: (0, 0)),
        ],
        out_specs=pl.BlockSpec((_RP, CH), lambda i: (i, 0)),
        out_shape=jax.ShapeDtypeStruct((PH, CH), jnp.float32),
    )(qa, qb, g2p, dvp, b2p)


def kernel(x, edge_index, W1, b1, W2, b2):
    src = edge_index[0].astype(jnp.int32)
    dst = edge_index[1].astype(jnp.int32)
    pad_src = (jnp.arange(PADE, dtype=jnp.int32) * 7919) % N
    pad_dst = jnp.full((PADE,), N, jnp.int32)
    src2 = jnp.concatenate([src, pad_src]).reshape(NCHUNK, CH)
    dst2 = jnp.concatenate([dst, pad_dst]).reshape(NCHUNK, CH)

    z1 = jnp.zeros((NSUB,), jnp.float32)
    z16 = jnp.zeros((NSUB, D_HID), jnp.float32)
    ones = jnp.ones((CH,), jnp.float32)

    dega, degb = _k_deg(dst2, z1, ones)
    dvp = _k_dinv(dega, degb)

    x3 = x.reshape(N // 8, 8, D_IN)
    g1p = _k_mm1(x3, W1, dvp)
    p1a, p1b = _k_agg(src2, dst2, g1p.reshape(NP, D_HID), z16)

    W2p = jnp.pad(W2, ((0, 0), (0, D_HID - N_CLS)))
    W2t = jnp.kron(jnp.eye(8, dtype=jnp.float32), W2p)
    b1p = jnp.tile(b1, 8).reshape(1, CH)
    b2p = jnp.tile(jnp.pad(b2, (0, D_HID - N_CLS)), 8).reshape(1, CH)

    g2p = _k_mm2(p1a.reshape(PH, CH), p1b.reshape(PH, CH), g1p, dvp, b1p, W2t)
    p2a, p2b = _k_agg(src2, dst2, g2p.reshape(NP, D_HID), z16)
    outp = _k_final(p2a.reshape(PH, CH), p2b.reshape(PH, CH), g2p, dvp, b2p)
    return outp.reshape(NP, D_HID)[:N, :N_CLS]


# speedup vs baseline: 83.8095x; 1.1284x over previous
import jax
import jax.numpy as jnp
from jax import lax
from jax.experimental import pallas as pl
from jax.experimental.pallas import tpu as pltpu
from jax.experimental.pallas import tpu_sc as plsc

N = 100000
E = 1600000
D_IN = 128
D_HID = 16
N_CLS = 7

NC = 2
NS = 16
NW = NC * NS

CH = 128
K = 4
M = 98
EW = K * M * CH
EP = EW * NW
NCHUNK = EP // CH
PADE = EP - E

NP = 100352
PH = NP // 8
NSUB = NP // NS
NWRK = NP // NW

_MESH = plsc.VectorSubcoreMesh(core_axis_name="c", subcore_axis_name="s")
_SC_PARAMS = pltpu.CompilerParams(use_tc_tiling_on_sc=False)


def _deg_body(dst_hbm, zer_hbm, one_hbm, dega_hbm, degb_hbm,
              deg_sh, idx_v, one_v, sem):
    c = lax.axis_index("c")
    s = lax.axis_index("s")
    off = s * NSUB
    pltpu.sync_copy(zer_hbm, deg_sh.at[pl.ds(off, NSUB)])
    pltpu.sync_copy(one_hbm, one_v)
    plsc.subcore_barrier()

    cb = (c * NS + s) * (M * K)
    pltpu.sync_copy(dst_hbm.at[pl.ds(cb, M * K)], idx_v)

    def t_body(t, carry):
        for k in range(K):
            pltpu.async_copy(one_v, deg_sh.at[idx_v.at[t * K + k]], sem,
                             add=True)

        @pl.when(t > 0)
        def _():
            for k in range(K):
                pltpu.make_async_copy(
                    one_v, deg_sh.at[idx_v.at[(t - 1) * K + k]], sem).wait()
        return carry
    lax.fori_loop(0, M, t_body, 0)
    for k in range(K):
        pltpu.make_async_copy(
            one_v, deg_sh.at[idx_v.at[(M - 1) * K + k]], sem).wait()

    plsc.subcore_barrier()

    @pl.when(c == 0)
    def _():
        pltpu.sync_copy(deg_sh.at[pl.ds(off, NSUB)], dega_hbm.at[pl.ds(off, NSUB)])

    @pl.when(c == 1)
    def _():
        pltpu.sync_copy(deg_sh.at[pl.ds(off, NSUB)], degb_hbm.at[pl.ds(off, NSUB)])


def _k_deg(dst2, z1, ones):
    f = pl.kernel(
        _deg_body,
        out_type=[jax.ShapeDtypeStruct((NP,), jnp.float32),
                  jax.ShapeDtypeStruct((NP,), jnp.float32)],
        mesh=_MESH,
        scratch_types=[
            pltpu.VMEM_SHARED((NP,), jnp.float32),
            pltpu.VMEM((M * K, CH), jnp.int32),
            pltpu.VMEM((CH,), jnp.float32),
            pltpu.SemaphoreType.DMA,
        ],
        compiler_params=_SC_PARAMS,
    )
    return f(dst2, z1, ones)


def _dinv_body(dega_hbm, degb_hbm, dv_hbm, va, vb, vo, stage):
    c = lax.axis_index("c")
    s = lax.axis_index("s")
    w = c * NS + s
    off = w * NWRK
    pltpu.sync_copy(dega_hbm.at[pl.ds(off, NWRK)], va)
    pltpu.sync_copy(degb_hbm.at[pl.ds(off, NWRK)], vb)

    def body(i, carry):
        d = va[pl.ds(i * 16, 16)] + vb[pl.ds(i * 16, 16)] + 1.0
        bits = lax.bitcast_convert_type(d, jnp.int32)
        y = lax.bitcast_convert_type(
            jnp.int32(0x5F3759DF) - lax.shift_right_logical(bits, 1), jnp.float32)
        for _ in range(3):
            y = y * (1.5 - 0.5 * d * y * y)
        vo[pl.ds(i * 16, 16)] = y
        return carry
    lax.fori_loop(0, NWRK // 16, body, 0)

    def row_body(i, carry):
        v = vo[pl.ds(i * 16, 16)]
        for u in range(16):
            stage[2 * i + u // 8, pl.ds((u % 8) * 16, 16)] = jnp.full(
                (16,), v[u], jnp.float32)
        return carry
    lax.fori_loop(0, NWRK // 16, row_body, 0)
    pltpu.sync_copy(stage, dv_hbm.at[pl.ds(w * (NWRK // 8), NWRK // 8)])


def _k_dinv(dega, degb):
    f = pl.kernel(
        _dinv_body,
        out_type=jax.ShapeDtypeStruct((PH, CH), jnp.float32),
        mesh=_MESH,
        scratch_types=[
            pltpu.VMEM((NWRK,), jnp.float32),
            pltpu.VMEM((NWRK,), jnp.float32),
            pltpu.VMEM((NWRK,), jnp.float32),
            pltpu.VMEM((NWRK // 8, CH), jnp.float32),
        ],
        compiler_params=_SC_PARAMS,
    )
    return f(dega, degb)


def _agg_body(src_hbm, dst_hbm, g_hbm, zer_hbm, pa_hbm, pb_hbm,
              out_sh, isv, idv, rows, isem, gs0, gs1, ss0, ss1):
    c = lax.axis_index("c")
    s = lax.axis_index("s")
    off = s * NSUB
    pltpu.sync_copy(zer_hbm, out_sh.at[pl.ds(off, NSUB)])
    plsc.subcore_barrier()

    cb = (c * NS + s) * (M * K)
    gsem = (gs0, gs1)
    ssem = (ss0, ss1)

    for p in range(3):
        pltpu.async_copy(src_hbm.at[pl.ds(cb + p * K, K)],
                         isv.at[pl.ds(p * K, K)], isem)
        pltpu.async_copy(dst_hbm.at[pl.ds(cb + p * K, K)],
                         idv.at[pl.ds(p * K, K)], isem)
    pltpu.make_async_copy(src_hbm.at[pl.ds(cb, K)],
                          isv.at[pl.ds(0, K)], isem).wait()
    pltpu.make_async_copy(dst_hbm.at[pl.ds(cb, K)],
                          idv.at[pl.ds(0, K)], isem).wait()
    for k in range(K):
        pltpu.async_copy(g_hbm.at[isv.at[k]], rows.at[k], gsem[0])

    def step(t, buf):
        nxt = 1 - buf
        bb = buf * K
        nb = nxt * K
        islot = ((t + 1) % 4) * K

        @pl.when(t >= 1)
        def _():
            for k in range(K):
                pltpu.make_async_copy(
                    rows.at[nb + k], out_sh.at[idv.at[nb + k]],
                    ssem[nxt]).wait()

        @pl.when(t + 3 < M)
        def _():
            pltpu.async_copy(src_hbm.at[pl.ds(cb + (t + 3) * K, K)],
                             isv.at[pl.ds(((t + 3) % 4) * K, K)], isem)
            pltpu.async_copy(dst_hbm.at[pl.ds(cb + (t + 3) * K, K)],
                             idv.at[pl.ds(((t + 3) % 4) * K, K)], isem)

        @pl.when(t + 1 < M)
        def _():
            pltpu.make_async_copy(src_hbm.at[pl.ds(cb, K)],
                                  isv.at[pl.ds(islot, K)], isem).wait()
            pltpu.make_async_copy(dst_hbm.at[pl.ds(cb, K)],
                                  idv.at[pl.ds(islot, K)], isem).wait()
            for k in range(K):
                pltpu.async_copy(g_hbm.at[isv.at[islot + k]],
                                 rows.at[nb + k], gsem[nxt])

        bslot = (t % 4) * K
        for k in range(K):
            pltpu.make_async_copy(g_hbm.at[isv.at[bslot + k]],
                                  rows.at[bb + k], gsem[buf]).wait()
        for k in range(K):
            pltpu.async_copy(rows.at[bb + k], out_sh.at[idv.at[bslot + k]],
                             ssem[buf], add=True)

    def pair_body(i, carry):
        step(2 * i, 0)
        step(2 * i + 1, 1)
        return carry
    lax.fori_loop(0, M // 2, pair_body, 0)
    lslot = ((M - 1) % 4) * K
    for k in range(K):
        pltpu.make_async_copy(rows.at[K + k], out_sh.at[idv.at[lslot + k]],
                              ssem[1]).wait()

    plsc.subcore_barrier()

    @pl.when(c == 0)
    def _():
        pltpu.sync_copy(out_sh.at[pl.ds(off, NSUB)], pa_hbm.at[pl.ds(off, NSUB)])

    @pl.when(c == 1)
    def _():
        pltpu.sync_copy(out_sh.at[pl.ds(off, NSUB)], pb_hbm.at[pl.ds(off, NSUB)])


def _k_agg(src2, dst2, g, z16):
    f = pl.kernel(
        _agg_body,
        out_type=[jax.ShapeDtypeStruct((NP, D_HID), jnp.float32),
                  jax.ShapeDtypeStruct((NP, D_HID), jnp.float32)],
        mesh=_MESH,
        scratch_types=[
            pltpu.VMEM_SHARED((NP, D_HID), jnp.float32),
            pltpu.VMEM((4 * K, CH), jnp.int32),
            pltpu.VMEM((4 * K, CH), jnp.int32),
            pltpu.VMEM((2 * K, CH, D_HID), jnp.float32),
            pltpu.SemaphoreType.DMA,
            pltpu.SemaphoreType.DMA,
            pltpu.SemaphoreType.DMA,
            pltpu.SemaphoreType.DMA,
            pltpu.SemaphoreType.DMA,
        ],
        compiler_params=_SC_PARAMS,
    )
    return f(src2, dst2, g, z16)


_R = 2048
_RP = _R // 8
_G = NP // _R


def _mm1_body(x_ref, w_ref, dvp_ref, o_ref):
    for u in range(8):
        h = jnp.dot(x_ref[:, u, :], w_ref[...],
                    preferred_element_type=jnp.float32)
        o_ref[:, u * D_HID:(u + 1) * D_HID] = (
            h * dvp_ref[:, u * D_HID:(u + 1) * D_HID])


def _k_mm1(x3, W1, dvp):
    return pl.pallas_call(
        _mm1_body,
        grid=(_G,),
        in_specs=[
            pl.BlockSpec((_RP, 8, D_IN), lambda i: (i, 0, 0)),
            pl.BlockSpec((D_IN, D_HID), lambda i: (0, 0)),
            pl.BlockSpec((_RP, CH), lambda i: (i, 0)),
        ],
        out_specs=pl.BlockSpec((_RP, CH), lambda i: (i, 0)),
        out_shape=jax.ShapeDtypeStruct((PH, CH), jnp.float32),
    )(x3, W1, dvp)


def _mm2_body(pa_ref, pb_ref, g1_ref, dvp_ref, b1_ref, w2t_ref, o_ref):
    aggp = pa_ref[...] + pb_ref[...] + g1_ref[...]
    dvp = dvp_ref[...]
    hp = jnp.maximum(aggp * dvp + b1_ref[...], 0.0)
    g2 = jnp.dot(hp, w2t_ref[...], preferred_element_type=jnp.float32)
    o_ref[...] = g2 * dvp


def _k_mm2(pa, pb, g1p, dvp, b1p, W2t):
    return pl.pallas_call(
        _mm2_body,
        grid=(_G,),
        in_specs=[
            pl.BlockSpec((_RP, CH), lambda i: (i, 0)),
            pl.BlockSpec((_RP, CH), lambda i: (i, 0)),
            pl.BlockSpec((_RP, CH), lambda i: (i, 0)),
            pl.BlockSpec((_RP, CH), lambda i: (i, 0)),
            pl.BlockSpec((1, CH), lambda i: (0, 0)),
            pl.BlockSpec((CH, CH), lambda i: (0, 0)),
        ],
        out_specs=pl.BlockSpec((_RP, CH), lambda i: (i, 0)),
        out_shape=jax.ShapeDtypeStruct((PH, CH), jnp.float32),
    )(pa, pb, g1p, dvp, b1p, W2t)


_C8 = 8 * N_CLS


def _fin_body(qa_ref, qb_ref, g2_ref, dvp_ref, b2_ref, o_ref):
    z = (qa_ref[...] + qb_ref[...] + g2_ref[...]) * dvp_ref[...] + b2_ref[...]
    col = lax.broadcasted_iota(jnp.int32, (_RP, CH), 1)
    valid = (col % D_HID) < N_CLS
    ex = jnp.where(valid, jnp.exp(z), 0.0)
    gi = lax.broadcasted_iota(jnp.int32, (CH, CH), 0) // D_HID
    gj = lax.broadcasted_iota(jnp.int32, (CH, CH), 1) // D_HID
    gmat = (gi == gj).astype(jnp.float32)
    ssum = jnp.dot(ex, gmat, preferred_element_type=jnp.float32,
                   precision=lax.Precision.HIGHEST)
    lsm = z - jnp.log(ssum)
    pi = lax.broadcasted_iota(jnp.int32, (CH, _C8), 0)
    pj = lax.broadcasted_iota(jnp.int32, (CH, _C8), 1)
    perm = (pi == (pj // N_CLS) * D_HID + pj % N_CLS).astype(jnp.float32)
    o_ref[...] = jnp.dot(lsm, perm, preferred_element_type=jnp.float32,
                         precision=lax.Precision.HIGHEST)


def _k_final(qa, qb, g2p, dvp, b2p):
    return pl.pallas_call(
        _fin_body,
        grid=(_G,),
        in_specs=[
            pl.BlockSpec((_RP, CH), lambda i: (i, 0)),
            pl.BlockSpec((_RP, CH), lambda i: (i, 0)),
            pl.BlockSpec((_RP, CH), lambda i: (i, 0)),
            pl.BlockSpec((_RP, CH), lambda i: (i, 0)),
            pl.BlockSpec((1, CH), lambda i: (0, 0)),
        ],
        out_specs=pl.BlockSpec((_RP, _C8), lambda i: (i, 0)),
        out_shape=jax.ShapeDtypeStruct((PH, _C8), jnp.float32),
    )(qa, qb, g2p, dvp, b2p)


def kernel(x, edge_index, W1, b1, W2, b2):
    src = edge_index[0].astype(jnp.int32)
    dst = edge_index[1].astype(jnp.int32)
    pad_src = (jnp.arange(PADE, dtype=jnp.int32) * 7919) % N
    pad_dst = jnp.full((PADE,), N, jnp.int32)
    src2 = jnp.concatenate([src, pad_src]).reshape(NCHUNK, CH)
    dst2 = jnp.concatenate([dst, pad_dst]).reshape(NCHUNK, CH)

    z1 = jnp.zeros((NSUB,), jnp.float32)
    z16 = jnp.zeros((NSUB, D_HID), jnp.float32)
    ones = jnp.ones((CH,), jnp.float32)

    dega, degb = _k_deg(dst2, z1, ones)
    dvp = _k_dinv(dega, degb)

    x3 = x.reshape(N // 8, 8, D_IN)
    g1p = _k_mm1(x3, W1, dvp)
    p1a, p1b = _k_agg(src2, dst2, g1p.reshape(NP, D_HID), z16)

    W2p = jnp.pad(W2, ((0, 0), (0, D_HID - N_CLS)))
    W2t = jnp.kron(jnp.eye(8, dtype=jnp.float32), W2p)
    b1p = jnp.tile(b1, 8).reshape(1, CH)
    b2p = jnp.tile(jnp.pad(b2, (0, D_HID - N_CLS)), 8).reshape(1, CH)

    g2p = _k_mm2(p1a.reshape(PH, CH), p1b.reshape(PH, CH), g1p, dvp, b1p, W2t)
    p2a, p2b = _k_agg(src2, dst2, g2p.reshape(NP, D_HID), z16)
    outp = _k_final(p2a.reshape(PH, CH), p2b.reshape(PH, CH), g2p, dvp, b2p)
    return outp.reshape(NP, N_CLS)[:N]
